# Initial kernel scaffold; baseline (speedup 1.0000x reference)
import jax
import jax.numpy as jnp
from jax.experimental import pallas as pl


def kernel(x, edge_index, W1, b1, gamma1, beta1, Wmu, bmu):
    raise NotImplementedError("write your pallas kernel here")


# speedup vs baseline: 15.9578x; 15.9578x over previous
import functools
import math

import jax
import jax.numpy as jnp
from jax import lax
from jax.experimental import pallas as pl
from jax.experimental.pallas import tpu as pltpu
from jax.experimental.pallas import tpu_sc as plsc

N = 10000
E = 320000
IN = 128
OUT = 64
HID = 2 * OUT
EPS = 1e-5
RS = 1.0 / math.sqrt(1.0 + EPS)

NC = 2
NS = 16
NW = NC * NS
W_EDGES = E // NW
CH = 128
NFULL = W_EDGES // CH
TAIL = W_EDGES - NFULL * CH
RPT = 1000
NPAD = 10240

BM = 1000


def _sc_mesh():
    return plsc.VectorSubcoreMesh(core_axis_name="c", subcore_axis_name="s")


def _sc_degree(dst):
    @functools.partial(
        pl.kernel,
        out_type=jax.ShapeDtypeStruct((NC * NPAD,), jnp.float32),
        mesh=_sc_mesh(),
        scratch_types=[
            pltpu.VMEM((CH,), jnp.int32),
            pltpu.VMEM((TAIL,), jnp.int32),
            pltpu.VMEM((CH,), jnp.float32),
            pltpu.VMEM((CH,), jnp.float32),
            pltpu.VMEM_SHARED((NPAD,), jnp.float32),
        ],
    )
    def deg_kernel(dst_hbm, out_hbm, dst_v, dstt_v, ones_v, zeros_v, acc):
        cid = lax.axis_index("c")
        sid = lax.axis_index("s")
        for i in range(CH // 16):
            ones_v[pl.ds(i * 16, 16)] = jnp.ones((16,), jnp.float32)
            zeros_v[pl.ds(i * 16, 16)] = jnp.zeros((16,), jnp.float32)

        base = sid * (NPAD // NS)
        for j in range(NPAD // NS // CH):
            pltpu.sync_copy(zeros_v, acc.at[pl.ds(base + j * CH, CH)])

        plsc.subcore_barrier()
        ebase = (cid * NS + sid) * W_EDGES

        def body(j, carry):
            b = pl.multiple_of(ebase + j * CH, 16)
            pltpu.sync_copy(dst_hbm.at[pl.ds(b, CH)], dst_v)
            pltpu.sync_copy(ones_v, acc.at[dst_v], add=True)
            return carry

        lax.fori_loop(0, NFULL, body, 0)
        bt = pl.multiple_of(ebase + NFULL * CH, 16)
        pltpu.sync_copy(dst_hbm.at[pl.ds(bt, TAIL)], dstt_v)
        pltpu.sync_copy(ones_v.at[pl.ds(0, TAIL)], acc.at[dstt_v], add=True)
        plsc.subcore_barrier()

        @pl.when(sid == 0)
        def _():
            pltpu.sync_copy(acc.at[pl.ds(0, NPAD)],
                            out_hbm.at[pl.ds(cid * NPAD, NPAD)])

    return deg_kernel(dst)


def _sc_edge_scatter(g, src, dst, d):
    @functools.partial(
        pl.kernel,
        out_type=jax.ShapeDtypeStruct((NC, N, d), jnp.float32),
        mesh=_sc_mesh(),
        scratch_types=[
            pltpu.VMEM((CH,), jnp.int32),
            pltpu.VMEM((CH,), jnp.int32),
            pltpu.VMEM((TAIL,), jnp.int32),
            pltpu.VMEM((TAIL,), jnp.int32),
            pltpu.VMEM((CH, d), jnp.float32),
            pltpu.VMEM((16, d), jnp.float32),
            pltpu.VMEM_SHARED((N, d), jnp.float32),
            pltpu.SemaphoreType.DMA,
        ],
    )
    def scat_kernel(g_hbm, src_hbm, dst_hbm, out_hbm,
                    src_v, dst_v, srct_v, dstt_v, rows_v, z16_v, acc, sem):
        cid = lax.axis_index("c")
        sid = lax.axis_index("s")
        for r in range(16):
            for c in range(d // 16):
                z16_v[r, pl.ds(c * 16, 16)] = jnp.zeros((16,), jnp.float32)

        @pl.when(sid < N // RPT)
        def _():
            rbase = sid * RPT
            for kk in range(RPT // 16):
                pltpu.sync_copy(z16_v, acc.at[pl.ds(rbase + kk * 16, 16)])
            rem = RPT - (RPT // 16) * 16
            if rem:
                pltpu.sync_copy(z16_v.at[pl.ds(0, rem)],
                                acc.at[pl.ds(rbase + RPT - rem, rem)])

        plsc.subcore_barrier()
        ebase = (cid * NS + sid) * W_EDGES

        def body(j, carry):
            b = pl.multiple_of(ebase + j * CH, 16)
            pltpu.sync_copy(src_hbm.at[pl.ds(b, CH)], src_v)
            pltpu.sync_copy(dst_hbm.at[pl.ds(b, CH)], dst_v)
            pltpu.async_copy(g_hbm.at[src_v], rows_v, sem).wait()
            pltpu.sync_copy(rows_v, acc.at[dst_v], add=True)
            return carry

        lax.fori_loop(0, NFULL, body, 0)
        bt = pl.multiple_of(ebase + NFULL * CH, 16)
        pltpu.sync_copy(src_hbm.at[pl.ds(bt, TAIL)], srct_v)
        pltpu.sync_copy(dst_hbm.at[pl.ds(bt, TAIL)], dstt_v)
        pltpu.async_copy(g_hbm.at[srct_v], z16_v, sem).wait()
        pltpu.sync_copy(z16_v, acc.at[dstt_v], add=True)
        plsc.subcore_barrier()

        @pl.when(sid < N // RPT)
        def _():
            rbase = sid * RPT
            pltpu.sync_copy(acc.at[pl.ds(rbase, RPT)],
                            out_hbm.at[cid, pl.ds(rbase, RPT)])

    return scat_kernel(g, src, dst)


def _tc_h1(x, w1, g1r):
    def body(x_ref, w_ref, g_ref, o_ref):
        s = g_ref[...] * RS
        o_ref[...] = jnp.dot(x_ref[...], w_ref[...] * s,
                             preferred_element_type=jnp.float32,
                             precision=lax.Precision.HIGHEST)

    return pl.pallas_call(
        body,
        grid=(N // BM,),
        in_specs=[
            pl.BlockSpec((BM, IN), lambda i: (i, 0)),
            pl.BlockSpec((IN, HID), lambda i: (0, 0)),
            pl.BlockSpec((1, HID), lambda i: (0, 0)),
        ],
        out_specs=pl.BlockSpec((BM, HID), lambda i: (i, 0)),
        out_shape=jax.ShapeDtypeStruct((N, HID), jnp.float32),
    )(x, w1, g1r)


def _tc_scale(dpt, h1):
    def body(dp_ref, h_ref, g_ref, d_ref):
        deg = dp_ref[:, 0:1] + dp_ref[:, 1:2] + 1.0
        dis = lax.rsqrt(deg)
        d_ref[...] = dis
        g_ref[...] = h_ref[...] * dis

    return pl.pallas_call(
        body,
        grid=(N // BM,),
        in_specs=[
            pl.BlockSpec((BM, NC), lambda i: (i, 0)),
            pl.BlockSpec((BM, HID), lambda i: (i, 0)),
        ],
        out_specs=[
            pl.BlockSpec((BM, HID), lambda i: (i, 0)),
            pl.BlockSpec((BM, 1), lambda i: (i, 0)),
        ],
        out_shape=[
            jax.ShapeDtypeStruct((N, HID), jnp.float32),
            jax.ShapeDtypeStruct((N, 1), jnp.float32),
        ],
    )(dpt, h1)


def _tc_combine1(p, g1, dis, b1r, g1r, bt1r):
    def body(p_ref, g1_ref, d_ref, b_ref, gm_ref, bt_ref, o_ref):
        dis = d_ref[...]
        pre = (p_ref[0] + p_ref[1] + g1_ref[...]) * dis
        h = jnp.maximum(pre + (b_ref[...] * (gm_ref[...] * RS) + bt_ref[...]),
                        0.0)
        o_ref[...] = h * dis

    return pl.pallas_call(
        body,
        grid=(N // BM,),
        in_specs=[
            pl.BlockSpec((NC, BM, HID), lambda i: (0, i, 0)),
            pl.BlockSpec((BM, HID), lambda i: (i, 0)),
            pl.BlockSpec((BM, 1), lambda i: (i, 0)),
            pl.BlockSpec((1, HID), lambda i: (0, 0)),
            pl.BlockSpec((1, HID), lambda i: (0, 0)),
            pl.BlockSpec((1, HID), lambda i: (0, 0)),
        ],
        out_specs=pl.BlockSpec((BM, HID), lambda i: (i, 0)),
        out_shape=jax.ShapeDtypeStruct((N, HID), jnp.float32),
    )(p, g1, dis, b1r, g1r, bt1r)


def _tc_combine2(q, gh, dis, wmu, bmur):
    def body(q_ref, gh_ref, d_ref, w_ref, b_ref, o_ref):
        z = (q_ref[0] + q_ref[1] + gh_ref[...]) * d_ref[...]
        o_ref[...] = (jnp.dot(z, w_ref[...], preferred_element_type=jnp.float32,
                              precision=lax.Precision.HIGHEST)
                      + b_ref[...])

    return pl.pallas_call(
        body,
        grid=(N // BM,),
        in_specs=[
            pl.BlockSpec((NC, BM, HID), lambda i: (0, i, 0)),
            pl.BlockSpec((BM, HID), lambda i: (i, 0)),
            pl.BlockSpec((BM, 1), lambda i: (i, 0)),
            pl.BlockSpec((HID, OUT), lambda i: (0, 0)),
            pl.BlockSpec((1, OUT), lambda i: (0, 0)),
        ],
        out_specs=pl.BlockSpec((BM, OUT), lambda i: (i, 0)),
        out_shape=jax.ShapeDtypeStruct((N, OUT), jnp.float32),
    )(q, gh, dis, wmu, bmur)


def kernel(x, edge_index, W1, b1, gamma1, beta1, Wmu, bmu):
    src = edge_index[0]
    dst = edge_index[1]
    g1r = gamma1.reshape(1, HID)
    b1r = b1.reshape(1, HID)
    bt1r = beta1.reshape(1, HID)
    bmur = bmu.reshape(1, OUT)

    degp = _sc_degree(dst).reshape(NC, NPAD)[:, :N]
    h1 = _tc_h1(x, W1, g1r)
    g1_arr, dis = _tc_scale(degp.T, h1)
    p = _sc_edge_scatter(g1_arr, src, dst, HID)
    gh = _tc_combine1(p, g1_arr, dis, b1r, g1r, bt1r)
    q = _sc_edge_scatter(gh, src, dst, HID)
    mu = _tc_combine2(q, gh, dis, Wmu, bmur)
    return (mu, mu, mu)


# speedup vs baseline: 25.0828x; 1.5718x over previous
import functools
import math

import jax
import jax.numpy as jnp
from jax import lax
from jax.experimental import pallas as pl
from jax.experimental.pallas import tpu as pltpu
from jax.experimental.pallas import tpu_sc as plsc

N = 10000
E = 320000
IN = 128
OUT = 64
HID = 2 * OUT
EPS = 1e-5
RS = 1.0 / math.sqrt(1.0 + EPS)

NC = 2
NS = 16
NW = NC * NS
W_EDGES = E // NW
CH = 128
NFULL = W_EDGES // CH
TAIL = W_EDGES - NFULL * CH
RPT = 1000
NPAD = 10240

BM = 1000


def _sc_mesh():
    return plsc.VectorSubcoreMesh(core_axis_name="c", subcore_axis_name="s")


def _sc_degree(dst):
    @functools.partial(
        pl.kernel,
        out_type=jax.ShapeDtypeStruct((NC * NPAD,), jnp.float32),
        mesh=_sc_mesh(),
        scratch_types=[
            pltpu.VMEM((CH,), jnp.int32),
            pltpu.VMEM((TAIL,), jnp.int32),
            pltpu.VMEM((CH,), jnp.float32),
            pltpu.VMEM((CH,), jnp.float32),
            pltpu.VMEM_SHARED((NPAD,), jnp.float32),
        ],
    )
    def deg_kernel(dst_hbm, out_hbm, dst_v, dstt_v, ones_v, zeros_v, acc):
        cid = lax.axis_index("c")
        sid = lax.axis_index("s")
        for i in range(CH // 16):
            ones_v[pl.ds(i * 16, 16)] = jnp.ones((16,), jnp.float32)
            zeros_v[pl.ds(i * 16, 16)] = jnp.zeros((16,), jnp.float32)

        base = sid * (NPAD // NS)
        for j in range(NPAD // NS // CH):
            pltpu.sync_copy(zeros_v, acc.at[pl.ds(base + j * CH, CH)])

        plsc.subcore_barrier()
        ebase = (cid * NS + sid) * W_EDGES

        def body(j, carry):
            b = pl.multiple_of(ebase + j * CH, 16)
            pltpu.sync_copy(dst_hbm.at[pl.ds(b, CH)], dst_v)
            pltpu.sync_copy(ones_v, acc.at[dst_v], add=True)
            return carry

        lax.fori_loop(0, NFULL, body, 0, unroll=2)
        bt = pl.multiple_of(ebase + NFULL * CH, 16)
        pltpu.sync_copy(dst_hbm.at[pl.ds(bt, TAIL)], dstt_v)
        pltpu.sync_copy(ones_v.at[pl.ds(0, TAIL)], acc.at[dstt_v], add=True)
        plsc.subcore_barrier()

        @pl.when(sid == 0)
        def _():
            pltpu.sync_copy(acc.at[pl.ds(0, NPAD)],
                            out_hbm.at[pl.ds(cid * NPAD, NPAD)])

    return deg_kernel(dst)


def _sc_edge_scatter(g, src, dst, d):
    ring = 2
    iters = NFULL // ring

    @functools.partial(
        pl.kernel,
        out_type=jax.ShapeDtypeStruct((NC, N, d), jnp.float32),
        mesh=_sc_mesh(),
        scratch_types=[
            pltpu.VMEM((CH,), jnp.int32),
            pltpu.VMEM((CH,), jnp.int32),
            pltpu.VMEM((CH,), jnp.int32),
            pltpu.VMEM((CH,), jnp.int32),
            pltpu.VMEM((TAIL,), jnp.int32),
            pltpu.VMEM((TAIL,), jnp.int32),
            pltpu.VMEM((CH, d), jnp.float32),
            pltpu.VMEM((CH, d), jnp.float32),
            pltpu.VMEM((16, d), jnp.float32),
            pltpu.VMEM_SHARED((N, d), jnp.float32),
            pltpu.SemaphoreType.DMA,
            pltpu.SemaphoreType.DMA,
            pltpu.SemaphoreType.DMA,
        ],
    )
    def scat_kernel(g_hbm, src_hbm, dst_hbm, out_hbm,
                    src0, src1, dst0, dst1, srct_v, dstt_v,
                    rows0, rows1, z16_v, acc, isem, gsem, ssem):
        srcs = (src0, src1)
        dsts = (dst0, dst1)
        rows = (rows0, rows1)
        cid = lax.axis_index("c")
        sid = lax.axis_index("s")
        for r in range(16):
            for c in range(d // 16):
                z16_v[r, pl.ds(c * 16, 16)] = jnp.zeros((16,), jnp.float32)

        @pl.when(sid < N // RPT)
        def _():
            rbase = sid * RPT
            for kk in range(RPT // 16):
                pltpu.sync_copy(z16_v, acc.at[pl.ds(rbase + kk * 16, 16)])
            rem = RPT - (RPT // 16) * 16
            if rem:
                pltpu.sync_copy(z16_v.at[pl.ds(0, rem)],
                                acc.at[pl.ds(rbase + RPT - rem, rem)])

        plsc.subcore_barrier()
        ebase = (cid * NS + sid) * W_EDGES

        def idx_start(slot, c):
            b = pl.multiple_of(ebase + c * CH, 16)
            pltpu.async_copy(src_hbm.at[pl.ds(b, CH)], srcs[slot], isem)
            pltpu.async_copy(dst_hbm.at[pl.ds(b, CH)], dsts[slot], isem)

        def idx_wait(slot):
            pltpu.make_async_copy(src_hbm.at[pl.ds(0, CH)], srcs[slot],
                                  isem).wait()
            pltpu.make_async_copy(dst_hbm.at[pl.ds(0, CH)], dsts[slot],
                                  isem).wait()

        for b in range(ring):
            idx_start(b, b)

        def body(t, carry):
            for b in range(ring):
                idx_wait(b)
                pltpu.async_copy(g_hbm.at[srcs[b]], rows[b], gsem)
            for b in range(ring):
                pltpu.make_async_copy(g_hbm.at[srcs[b]], rows[b], gsem).wait()
                pltpu.async_copy(rows[b], acc.at[dsts[b]], ssem, add=True)
            for b in range(ring):
                pltpu.make_async_copy(rows[b], acc.at[dsts[b]], ssem).wait()

                @pl.when(t < iters - 1)
                def _():
                    idx_start(b, ring * (t + 1) + b)

            return carry

        lax.fori_loop(0, iters, body, 0)
        bt = pl.multiple_of(ebase + NFULL * CH, 16)
        pltpu.sync_copy(src_hbm.at[pl.ds(bt, TAIL)], srct_v)
        pltpu.sync_copy(dst_hbm.at[pl.ds(bt, TAIL)], dstt_v)
        pltpu.async_copy(g_hbm.at[srct_v], z16_v, gsem).wait()
        pltpu.sync_copy(z16_v, acc.at[dstt_v], add=True)
        plsc.subcore_barrier()

        @pl.when(sid < N // RPT)
        def _():
            rbase = sid * RPT
            pltpu.sync_copy(acc.at[pl.ds(rbase, RPT)],
                            out_hbm.at[cid, pl.ds(rbase, RPT)])

    return scat_kernel(g, src, dst)


def _tc_h1(x, w1, g1r):
    def body(x_ref, w_ref, g_ref, o_ref):
        s = g_ref[...] * RS
        o_ref[...] = jnp.dot(x_ref[...], w_ref[...] * s,
                             preferred_element_type=jnp.float32,
                             precision=lax.Precision.HIGHEST)

    return pl.pallas_call(
        body,
        grid=(N // BM,),
        in_specs=[
            pl.BlockSpec((BM, IN), lambda i: (i, 0)),
            pl.BlockSpec((IN, HID), lambda i: (0, 0)),
            pl.BlockSpec((1, HID), lambda i: (0, 0)),
        ],
        out_specs=pl.BlockSpec((BM, HID), lambda i: (i, 0)),
        out_shape=jax.ShapeDtypeStruct((N, HID), jnp.float32),
    )(x, w1, g1r)


def _tc_scale(dpt, h1):
    def body(dp_ref, h_ref, g_ref, d_ref):
        deg = dp_ref[:, 0:1] + dp_ref[:, 1:2] + 1.0
        dis = lax.rsqrt(deg)
        d_ref[...] = dis
        g_ref[...] = h_ref[...] * dis

    return pl.pallas_call(
        body,
        grid=(N // BM,),
        in_specs=[
            pl.BlockSpec((BM, NC), lambda i: (i, 0)),
            pl.BlockSpec((BM, HID), lambda i: (i, 0)),
        ],
        out_specs=[
            pl.BlockSpec((BM, HID), lambda i: (i, 0)),
            pl.BlockSpec((BM, 1), lambda i: (i, 0)),
        ],
        out_shape=[
            jax.ShapeDtypeStruct((N, HID), jnp.float32),
            jax.ShapeDtypeStruct((N, 1), jnp.float32),
        ],
    )(dpt, h1)


def _tc_combine1(p, g1, dis, b1r, g1r, bt1r):
    def body(p_ref, g1_ref, d_ref, b_ref, gm_ref, bt_ref, o_ref):
        dis = d_ref[...]
        pre = (p_ref[0] + p_ref[1] + g1_ref[...]) * dis
        h = jnp.maximum(pre + (b_ref[...] * (gm_ref[...] * RS) + bt_ref[...]),
                        0.0)
        o_ref[...] = h * dis

    return pl.pallas_call(
        body,
        grid=(N // BM,),
        in_specs=[
            pl.BlockSpec((NC, BM, HID), lambda i: (0, i, 0)),
            pl.BlockSpec((BM, HID), lambda i: (i, 0)),
            pl.BlockSpec((BM, 1), lambda i: (i, 0)),
            pl.BlockSpec((1, HID), lambda i: (0, 0)),
            pl.BlockSpec((1, HID), lambda i: (0, 0)),
            pl.BlockSpec((1, HID), lambda i: (0, 0)),
        ],
        out_specs=pl.BlockSpec((BM, HID), lambda i: (i, 0)),
        out_shape=jax.ShapeDtypeStruct((N, HID), jnp.float32),
    )(p, g1, dis, b1r, g1r, bt1r)


def _tc_combine2(q, gh, dis, wmu, bmur):
    def body(q_ref, gh_ref, d_ref, w_ref, b_ref, o_ref):
        z = (q_ref[0] + q_ref[1] + gh_ref[...]) * d_ref[...]
        o_ref[...] = (jnp.dot(z, w_ref[...], preferred_element_type=jnp.float32,
                              precision=lax.Precision.HIGHEST)
                      + b_ref[...])

    return pl.pallas_call(
        body,
        grid=(N // BM,),
        in_specs=[
            pl.BlockSpec((NC, BM, HID), lambda i: (0, i, 0)),
            pl.BlockSpec((BM, HID), lambda i: (i, 0)),
            pl.BlockSpec((BM, 1), lambda i: (i, 0)),
            pl.BlockSpec((HID, OUT), lambda i: (0, 0)),
            pl.BlockSpec((1, OUT), lambda i: (0, 0)),
        ],
        out_specs=pl.BlockSpec((BM, OUT), lambda i: (i, 0)),
        out_shape=jax.ShapeDtypeStruct((N, OUT), jnp.float32),
    )(q, gh, dis, wmu, bmur)


def kernel(x, edge_index, W1, b1, gamma1, beta1, Wmu, bmu):
    src = edge_index[0]
    dst = edge_index[1]
    g1r = gamma1.reshape(1, HID)
    b1r = b1.reshape(1, HID)
    bt1r = beta1.reshape(1, HID)
    bmur = bmu.reshape(1, OUT)

    degp = _sc_degree(dst).reshape(NC, NPAD)[:, :N]
    h1 = _tc_h1(x, W1, g1r)
    g1_arr, dis = _tc_scale(degp.T, h1)
    p = _sc_edge_scatter(g1_arr, src, dst, HID)
    gh = _tc_combine1(p, g1_arr, dis, b1r, g1r, bt1r)
    q = _sc_edge_scatter(gh, src, dst, HID)
    mu = _tc_combine2(q, gh, dis, Wmu, bmur)
    return (mu, mu, mu)
